# Initial kernel scaffold; baseline (speedup 1.0000x reference)
import jax
import jax.numpy as jnp
from jax.experimental import pallas as pl


def kernel(x_user, x_item, edge_index_u2i, edge_index_i2u, W1_u2i, W1_i2u, W1_root_user, W1_root_item, W2_u2i, W2_i2u, W2_root_user, W2_root_item):
    raise NotImplementedError("write your pallas kernel here")


# speedup vs baseline: 2.3763x; 2.3763x over previous
import functools

import jax
import jax.numpy as jnp
from jax import lax
from jax.experimental import pallas as pl
from jax.experimental.pallas import tpu as pltpu
from jax.experimental.pallas import tpu_sc as plsc

N = 10000
D = 128
E = 320000
NC = 2
NS = 16
L = 16

CH = 128
CPT = 157
PER_TILE = CPT * CH
EP = NS * PER_TILE
ACC = 10240
DUMMY = N
OPT = ACC // NS

BLK = 1000
NBLK = N // BLK


def _sc_kernel(with_counts):
    mesh = plsc.VectorSubcoreMesh(core_axis_name="c", subcore_axis_name="s",
                                  num_cores=NC, num_subcores=NS)
    out_type = [jax.ShapeDtypeStruct((2, ACC, D), jnp.float32)]
    if with_counts:
        out_type.append(jax.ShapeDtypeStruct((2, ACC, D), jnp.float32))
    scratch = [
        pltpu.VMEM((CH,), jnp.int32),
        pltpu.VMEM((CH,), jnp.int32),
        pltpu.VMEM((CH, D), jnp.float32),
        pltpu.VMEM_SHARED((ACC, D), jnp.float32),
        pltpu.SemaphoreType.DMA,
    ]

    def body(*refs):
        if with_counts:
            (src_hbm, dst_hbm, m_hbm, zrow_hbm, ones_hbm,
             sums_out, counts_out,
             sidx, didx, rows, acc_sh, sem) = refs
        else:
            (src_hbm, dst_hbm, m_hbm, zrow_hbm, ones_hbm,
             sums_out,
             sidx, didx, rows, acc_sh, sem) = refs
        c = lax.axis_index("c")
        t = lax.axis_index("s")
        on0 = c == 0

        def phase(d, out_ref, counting):
            @pl.when(on0)
            def _():
                pltpu.sync_copy(zrow_hbm, rows)
                for i in range(ACC // CH // NS):
                    blk = i * NS + t
                    pltpu.sync_copy(rows, acc_sh.at[pl.ds(blk * CH, CH)])
            plsc.subcore_barrier()

            @pl.when(on0)
            def _():
                if counting:
                    pltpu.sync_copy(ones_hbm, rows)
                base = d * EP + t * PER_TILE

                def step(g, _):
                    eb = pl.multiple_of(base + g * CH, CH)
                    pltpu.sync_copy(dst_hbm.at[pl.ds(eb, CH)], didx)
                    if not counting:
                        pltpu.sync_copy(src_hbm.at[pl.ds(eb, CH)], sidx)
                        pltpu.async_copy(m_hbm.at[sidx], rows, sem).wait()
                    pltpu.sync_copy(rows, acc_sh.at[didx], add=True)
                    return _
                lax.fori_loop(0, CPT, step, None)
            plsc.subcore_barrier()

            @pl.when(on0)
            def _():
                ob = t * OPT
                pltpu.sync_copy(acc_sh.at[pl.ds(ob, OPT)],
                                out_ref.at[d, pl.ds(ob, OPT)])
            plsc.subcore_barrier()

        phase(0, sums_out, False)
        phase(1, sums_out, False)
        if with_counts:
            phase(0, counts_out, True)
            phase(1, counts_out, True)

    return pl.kernel(body, out_type=out_type, mesh=mesh, scratch_types=scratch)


def _mm_body(x_ref, wm_ref, wr_ref, m_ref, r_ref):
    x = x_ref[...]
    m_ref[...] = jnp.dot(x, wm_ref[0], preferred_element_type=jnp.float32)
    r_ref[...] = jnp.dot(x, wr_ref[0], preferred_element_type=jnp.float32)


def _mm(xcat, Wm, Wr):
    return pl.pallas_call(
        _mm_body,
        grid=(2, NBLK),
        in_specs=[
            pl.BlockSpec((BLK, D), lambda t, b: (t * NBLK + b, 0)),
            pl.BlockSpec((1, D, D), lambda t, b: (t, 0, 0)),
            pl.BlockSpec((1, D, D), lambda t, b: (t, 0, 0)),
        ],
        out_specs=[
            pl.BlockSpec((BLK, D), lambda t, b: (t * NBLK + b, 0)),
            pl.BlockSpec((BLK, D), lambda t, b: (t * NBLK + b, 0)),
        ],
        out_shape=[
            jax.ShapeDtypeStruct((2 * N, D), jnp.float32),
            jax.ShapeDtypeStruct((2 * N, D), jnp.float32),
        ],
    )(xcat, Wm, Wr)


def _combine_body(sums_ref, counts_ref, root_ref, out_ref, *, act):
    cnt = counts_ref[0, :, 0:1]
    h = sums_ref[0] / jnp.maximum(cnt, 1.0) + root_ref[...]
    if act:
        h = jnp.maximum(h, 0.0)
    out_ref[...] = h


def _combine(sums, counts, root, act):
    return pl.pallas_call(
        functools.partial(_combine_body, act=act),
        grid=(2, NBLK),
        in_specs=[
            pl.BlockSpec((1, BLK, D), lambda t, b: (1 - t, b, 0)),
            pl.BlockSpec((1, BLK, D), lambda t, b: (1 - t, b, 0)),
            pl.BlockSpec((BLK, D), lambda t, b: (t * NBLK + b, 0)),
        ],
        out_specs=pl.BlockSpec((BLK, D), lambda t, b: (t * NBLK + b, 0)),
        out_shape=jax.ShapeDtypeStruct((2 * N, D), jnp.float32),
    )(sums, counts, root)


def kernel(x_user, x_item, edge_index_u2i, edge_index_i2u,
           W1_u2i, W1_i2u, W1_root_user, W1_root_item,
           W2_u2i, W2_i2u, W2_root_user, W2_root_item):
    ei_u2i = edge_index_u2i.astype(jnp.int32)
    ei_i2u = edge_index_i2u.astype(jnp.int32)
    pad = EP - E
    src_all = jnp.concatenate([
        ei_u2i[0], jnp.zeros((pad,), jnp.int32),
        ei_i2u[0] + N, jnp.full((pad,), N, jnp.int32),
    ])
    dpad = jnp.full((pad,), DUMMY, jnp.int32)
    dst_all = jnp.concatenate([ei_u2i[1], dpad, ei_i2u[1], dpad])

    zrow = jnp.zeros((CH, D), jnp.float32)
    ones = jnp.ones((CH, D), jnp.float32)

    xcat = jnp.concatenate([x_user, x_item], axis=0)
    W1m = jnp.stack([W1_u2i, W1_i2u])
    W1r = jnp.stack([W1_root_user, W1_root_item])
    W2m = jnp.stack([W2_u2i, W2_i2u])
    W2r = jnp.stack([W2_root_user, W2_root_item])

    m1, r1 = _mm(xcat, W1m, W1r)
    sums1, counts = _sc_kernel(True)(src_all, dst_all, m1, zrow, ones)
    h1 = _combine(sums1, counts, r1, act=True)
    m2, r2 = _mm(h1, W2m, W2r)
    (sums2,) = _sc_kernel(False)(src_all, dst_all, m2, zrow, ones)
    out = _combine(sums2, counts, r2, act=False)
    return out


# speedup vs baseline: 2.5567x; 1.0759x over previous
import functools

import jax
import jax.numpy as jnp
from jax import lax
from jax.experimental import pallas as pl
from jax.experimental.pallas import tpu as pltpu
from jax.experimental.pallas import tpu_sc as plsc

N = 10000
D = 128
E = 320000
NC = 2
NS = 16
L = 16

CH = 64
CPT = 314
KP = CPT // 2
PER_TILE = CPT * CH
EP = NS * PER_TILE
ACC = 10240
DUMMY = N
OPT = ACC // NS

BLK = 1000
NBLK = N // BLK


def _sc_kernel(with_counts):
    mesh = plsc.VectorSubcoreMesh(core_axis_name="c", subcore_axis_name="s",
                                  num_cores=NC, num_subcores=NS)
    out_type = [jax.ShapeDtypeStruct((2, ACC, D), jnp.float32)]
    if with_counts:
        out_type.append(jax.ShapeDtypeStruct((2, ACC, D), jnp.float32))
    scratch = [
        pltpu.VMEM((CH,), jnp.int32),
        pltpu.VMEM((CH,), jnp.int32),
        pltpu.VMEM((CH,), jnp.int32),
        pltpu.VMEM((CH,), jnp.int32),
        pltpu.VMEM((CH, D), jnp.float32),
        pltpu.VMEM((CH, D), jnp.float32),
        pltpu.VMEM_SHARED((ACC, D), jnp.float32),
        pltpu.SemaphoreType.DMA,
        pltpu.SemaphoreType.DMA,
    ]

    def body(*refs):
        if with_counts:
            (src_hbm, dst_hbm, m_hbm, zrow_hbm, ones_hbm,
             sums_out, counts_out,
             sidx0, sidx1, didx0, didx1, rows0, rows1,
             acc_sh, sem0, sem1) = refs
        else:
            (src_hbm, dst_hbm, m_hbm, zrow_hbm, ones_hbm,
             sums_out,
             sidx0, sidx1, didx0, didx1, rows0, rows1,
             acc_sh, sem0, sem1) = refs
        c = lax.axis_index("c")
        t = lax.axis_index("s")
        on0 = c == 0

        def phase(d, out_ref, counting):
            @pl.when(on0)
            def _():
                pltpu.sync_copy(zrow_hbm, rows0)
                for i in range(ACC // CH // NS):
                    blk = i * NS + t
                    pltpu.sync_copy(rows0, acc_sh.at[pl.ds(blk * CH, CH)])
            plsc.subcore_barrier()

            @pl.when(on0)
            def _():
                base = d * EP + t * PER_TILE

                def eoff(g):
                    return pl.multiple_of(base + g * CH, CH)

                if counting:
                    pltpu.sync_copy(ones_hbm, rows0)
                    pltpu.sync_copy(dst_hbm.at[pl.ds(eoff(0), CH)], didx0)

                    def cstep(k, _):
                        pltpu.sync_copy(dst_hbm.at[pl.ds(eoff(2 * k + 1), CH)],
                                        didx1)
                        pltpu.sync_copy(rows0, acc_sh.at[didx0], add=True)

                        @pl.when(k < KP - 1)
                        def _():
                            pltpu.sync_copy(
                                dst_hbm.at[pl.ds(eoff(2 * k + 2), CH)], didx0)
                        pltpu.sync_copy(rows0, acc_sh.at[didx1], add=True)
                        return _
                    lax.fori_loop(0, KP, cstep, None)
                else:
                    pltpu.sync_copy(src_hbm.at[pl.ds(eoff(0), CH)], sidx0)
                    pltpu.sync_copy(dst_hbm.at[pl.ds(eoff(0), CH)], didx0)
                    cp0 = pltpu.async_copy(m_hbm.at[sidx0], rows0, sem0)

                    def step(k, _):
                        pltpu.sync_copy(src_hbm.at[pl.ds(eoff(2 * k + 1), CH)],
                                        sidx1)
                        pltpu.sync_copy(dst_hbm.at[pl.ds(eoff(2 * k + 1), CH)],
                                        didx1)
                        cp1 = pltpu.async_copy(m_hbm.at[sidx1], rows1, sem1)
                        pltpu.make_async_copy(m_hbm.at[sidx0], rows0,
                                              sem0).wait()
                        pltpu.sync_copy(rows0, acc_sh.at[didx0], add=True)

                        @pl.when(k < KP - 1)
                        def _():
                            pltpu.sync_copy(
                                src_hbm.at[pl.ds(eoff(2 * k + 2), CH)], sidx0)
                            pltpu.sync_copy(
                                dst_hbm.at[pl.ds(eoff(2 * k + 2), CH)], didx0)
                            pltpu.async_copy(m_hbm.at[sidx0], rows0, sem0)
                        cp1.wait()
                        pltpu.sync_copy(rows1, acc_sh.at[didx1], add=True)
                        return _
                    lax.fori_loop(0, KP, step, None)
            plsc.subcore_barrier()

            @pl.when(on0)
            def _():
                ob = t * OPT
                pltpu.sync_copy(acc_sh.at[pl.ds(ob, OPT)],
                                out_ref.at[d, pl.ds(ob, OPT)])
            plsc.subcore_barrier()

        phase(0, sums_out, False)
        phase(1, sums_out, False)
        if with_counts:
            phase(0, counts_out, True)
            phase(1, counts_out, True)

    return pl.kernel(body, out_type=out_type, mesh=mesh, scratch_types=scratch)


def _mm_body(x_ref, wm_ref, wr_ref, m_ref, r_ref):
    x = x_ref[...]
    m_ref[...] = jnp.dot(x, wm_ref[0], preferred_element_type=jnp.float32)
    r_ref[...] = jnp.dot(x, wr_ref[0], preferred_element_type=jnp.float32)


def _mm(xcat, Wm, Wr):
    return pl.pallas_call(
        _mm_body,
        grid=(2, NBLK),
        in_specs=[
            pl.BlockSpec((BLK, D), lambda t, b: (t * NBLK + b, 0)),
            pl.BlockSpec((1, D, D), lambda t, b: (t, 0, 0)),
            pl.BlockSpec((1, D, D), lambda t, b: (t, 0, 0)),
        ],
        out_specs=[
            pl.BlockSpec((BLK, D), lambda t, b: (t * NBLK + b, 0)),
            pl.BlockSpec((BLK, D), lambda t, b: (t * NBLK + b, 0)),
        ],
        out_shape=[
            jax.ShapeDtypeStruct((2 * N, D), jnp.float32),
            jax.ShapeDtypeStruct((2 * N, D), jnp.float32),
        ],
    )(xcat, Wm, Wr)


def _combine_body(sums_ref, counts_ref, root_ref, out_ref, *, act):
    cnt = counts_ref[0, :, 0:1]
    h = sums_ref[0] / jnp.maximum(cnt, 1.0) + root_ref[...]
    if act:
        h = jnp.maximum(h, 0.0)
    out_ref[...] = h


def _combine(sums, counts, root, act):
    return pl.pallas_call(
        functools.partial(_combine_body, act=act),
        grid=(2, NBLK),
        in_specs=[
            pl.BlockSpec((1, BLK, D), lambda t, b: (1 - t, b, 0)),
            pl.BlockSpec((1, BLK, D), lambda t, b: (1 - t, b, 0)),
            pl.BlockSpec((BLK, D), lambda t, b: (t * NBLK + b, 0)),
        ],
        out_specs=pl.BlockSpec((BLK, D), lambda t, b: (t * NBLK + b, 0)),
        out_shape=jax.ShapeDtypeStruct((2 * N, D), jnp.float32),
    )(sums, counts, root)


def kernel(x_user, x_item, edge_index_u2i, edge_index_i2u,
           W1_u2i, W1_i2u, W1_root_user, W1_root_item,
           W2_u2i, W2_i2u, W2_root_user, W2_root_item):
    ei_u2i = edge_index_u2i.astype(jnp.int32)
    ei_i2u = edge_index_i2u.astype(jnp.int32)
    pad = EP - E
    src_all = jnp.concatenate([
        ei_u2i[0], jnp.zeros((pad,), jnp.int32),
        ei_i2u[0] + N, jnp.full((pad,), N, jnp.int32),
    ])
    dpad = jnp.full((pad,), DUMMY, jnp.int32)
    dst_all = jnp.concatenate([ei_u2i[1], dpad, ei_i2u[1], dpad])

    zrow = jnp.zeros((CH, D), jnp.float32)
    ones = jnp.ones((CH, D), jnp.float32)

    xcat = jnp.concatenate([x_user, x_item], axis=0)
    W1m = jnp.stack([W1_u2i, W1_i2u])
    W1r = jnp.stack([W1_root_user, W1_root_item])
    W2m = jnp.stack([W2_u2i, W2_i2u])
    W2r = jnp.stack([W2_root_user, W2_root_item])

    m1, r1 = _mm(xcat, W1m, W1r)
    sums1, counts = _sc_kernel(True)(src_all, dst_all, m1, zrow, ones)
    h1 = _combine(sums1, counts, r1, act=True)
    m2, r2 = _mm(h1, W2m, W2r)
    (sums2,) = _sc_kernel(False)(src_all, dst_all, m2, zrow, ones)
    out = _combine(sums2, counts, r2, act=False)
    return out


# speedup vs baseline: 2.8203x; 1.1031x over previous
import functools

import jax
import jax.numpy as jnp
from jax import lax
from jax.experimental import pallas as pl
from jax.experimental.pallas import tpu as pltpu
from jax.experimental.pallas import tpu_sc as plsc

N = 10000
D = 128
E = 320000
NC = 2
NS = 16
L = 16

CH = 64
CPT = 314
KP = CPT // 2
PER_TILE = CPT * CH
EP = NS * PER_TILE
ACC = 10240
DUMMY = N
OPT = ACC // NS

BLK = 1000
NBLK = N // BLK


def _sc_kernel(with_counts):
    mesh = plsc.VectorSubcoreMesh(core_axis_name="c", subcore_axis_name="s",
                                  num_cores=NC, num_subcores=NS)
    out_type = [jax.ShapeDtypeStruct((2, ACC, D), jnp.float32)]
    if with_counts:
        out_type.append(jax.ShapeDtypeStruct((2, ACC, D), jnp.float32))
    scratch = [
        pltpu.VMEM((CH,), jnp.int32),
        pltpu.VMEM((CH,), jnp.int32),
        pltpu.VMEM((CH,), jnp.int32),
        pltpu.VMEM((CH,), jnp.int32),
        pltpu.VMEM((CH, D), jnp.float32),
        pltpu.VMEM((CH, D), jnp.float32),
        pltpu.VMEM_SHARED((ACC, D), jnp.float32),
        pltpu.SemaphoreType.DMA,
        pltpu.SemaphoreType.DMA,
    ]

    def body(*refs):
        if with_counts:
            (src_hbm, dst_hbm, m_hbm, zrow_hbm, ones_hbm,
             sums_out, counts_out,
             sidx0, sidx1, didx0, didx1, rows0, rows1,
             acc_sh, sem0, sem1) = refs
        else:
            (src_hbm, dst_hbm, m_hbm, zrow_hbm, ones_hbm,
             sums_out,
             sidx0, sidx1, didx0, didx1, rows0, rows1,
             acc_sh, sem0, sem1) = refs
        c = lax.axis_index("c")
        t = lax.axis_index("s")
        on0 = c == 0

        def phase(d, out_ref, counting):
            @pl.when(on0)
            def _():
                pltpu.sync_copy(zrow_hbm, rows0)
                for i in range(ACC // CH // NS):
                    blk = i * NS + t
                    pltpu.sync_copy(rows0, acc_sh.at[pl.ds(blk * CH, CH)])
            plsc.subcore_barrier()

            @pl.when(on0)
            def _():
                base = d * EP + t * PER_TILE

                def eoff(g):
                    return pl.multiple_of(base + g * CH, CH)

                if counting:
                    pltpu.sync_copy(ones_hbm, rows0)
                    pltpu.sync_copy(dst_hbm.at[pl.ds(eoff(0), CH)], didx0)

                    def cstep(k, _):
                        c0 = pltpu.async_copy(rows0, acc_sh.at[didx0], sem0,
                                              add=True)
                        pltpu.sync_copy(dst_hbm.at[pl.ds(eoff(2 * k + 1), CH)],
                                        didx1)
                        c1 = pltpu.async_copy(rows0, acc_sh.at[didx1], sem1,
                                              add=True)
                        c0.wait()

                        @pl.when(k < KP - 1)
                        def _():
                            pltpu.sync_copy(
                                dst_hbm.at[pl.ds(eoff(2 * k + 2), CH)], didx0)
                        c1.wait()
                        return _
                    lax.fori_loop(0, KP, cstep, None)
                else:
                    pltpu.sync_copy(src_hbm.at[pl.ds(eoff(0), CH)], sidx0)
                    pltpu.sync_copy(dst_hbm.at[pl.ds(eoff(0), CH)], didx0)
                    cp0 = pltpu.async_copy(m_hbm.at[sidx0], rows0, sem0)

                    def step(k, _):
                        pltpu.sync_copy(src_hbm.at[pl.ds(eoff(2 * k + 1), CH)],
                                        sidx1)
                        pltpu.sync_copy(dst_hbm.at[pl.ds(eoff(2 * k + 1), CH)],
                                        didx1)
                        cp1 = pltpu.async_copy(m_hbm.at[sidx1], rows1, sem1)
                        pltpu.make_async_copy(m_hbm.at[sidx0], rows0,
                                              sem0).wait()
                        pltpu.sync_copy(rows0, acc_sh.at[didx0], add=True)

                        @pl.when(k < KP - 1)
                        def _():
                            pltpu.sync_copy(
                                src_hbm.at[pl.ds(eoff(2 * k + 2), CH)], sidx0)
                            pltpu.sync_copy(
                                dst_hbm.at[pl.ds(eoff(2 * k + 2), CH)], didx0)
                            pltpu.async_copy(m_hbm.at[sidx0], rows0, sem0)
                        cp1.wait()
                        pltpu.sync_copy(rows1, acc_sh.at[didx1], add=True)
                        return _
                    lax.fori_loop(0, KP, step, None)
            plsc.subcore_barrier()

            @pl.when(on0)
            def _():
                ob = t * OPT
                pltpu.sync_copy(acc_sh.at[pl.ds(ob, OPT)],
                                out_ref.at[d, pl.ds(ob, OPT)])
            plsc.subcore_barrier()

        phase(0, sums_out, False)
        phase(1, sums_out, False)
        if with_counts:
            phase(0, counts_out, True)
            phase(1, counts_out, True)

    return pl.kernel(body, out_type=out_type, mesh=mesh, scratch_types=scratch)


def _mm_body(x_ref, wm_ref, wr_ref, m_ref, r_ref):
    x = x_ref[...]
    m_ref[...] = jnp.dot(x, wm_ref[0], preferred_element_type=jnp.float32)
    r_ref[...] = jnp.dot(x, wr_ref[0], preferred_element_type=jnp.float32)


def _mm(xcat, Wm, Wr):
    return pl.pallas_call(
        _mm_body,
        grid=(2, NBLK),
        in_specs=[
            pl.BlockSpec((BLK, D), lambda t, b: (t * NBLK + b, 0)),
            pl.BlockSpec((1, D, D), lambda t, b: (t, 0, 0)),
            pl.BlockSpec((1, D, D), lambda t, b: (t, 0, 0)),
        ],
        out_specs=[
            pl.BlockSpec((BLK, D), lambda t, b: (t * NBLK + b, 0)),
            pl.BlockSpec((BLK, D), lambda t, b: (t * NBLK + b, 0)),
        ],
        out_shape=[
            jax.ShapeDtypeStruct((2 * N, D), jnp.float32),
            jax.ShapeDtypeStruct((2 * N, D), jnp.float32),
        ],
    )(xcat, Wm, Wr)


def _combine_body(sums_ref, counts_ref, root_ref, out_ref, *, act):
    cnt = counts_ref[0, :, 0:1]
    h = sums_ref[0] / jnp.maximum(cnt, 1.0) + root_ref[...]
    if act:
        h = jnp.maximum(h, 0.0)
    out_ref[...] = h


def _combine(sums, counts, root, act):
    return pl.pallas_call(
        functools.partial(_combine_body, act=act),
        grid=(2, NBLK),
        in_specs=[
            pl.BlockSpec((1, BLK, D), lambda t, b: (1 - t, b, 0)),
            pl.BlockSpec((1, BLK, D), lambda t, b: (1 - t, b, 0)),
            pl.BlockSpec((BLK, D), lambda t, b: (t * NBLK + b, 0)),
        ],
        out_specs=pl.BlockSpec((BLK, D), lambda t, b: (t * NBLK + b, 0)),
        out_shape=jax.ShapeDtypeStruct((2 * N, D), jnp.float32),
    )(sums, counts, root)


def kernel(x_user, x_item, edge_index_u2i, edge_index_i2u,
           W1_u2i, W1_i2u, W1_root_user, W1_root_item,
           W2_u2i, W2_i2u, W2_root_user, W2_root_item):
    ei_u2i = edge_index_u2i.astype(jnp.int32)
    ei_i2u = edge_index_i2u.astype(jnp.int32)
    pad = EP - E
    src_all = jnp.concatenate([
        ei_u2i[0], jnp.zeros((pad,), jnp.int32),
        ei_i2u[0] + N, jnp.full((pad,), N, jnp.int32),
    ])
    dpad = jnp.full((pad,), DUMMY, jnp.int32)
    dst_all = jnp.concatenate([ei_u2i[1], dpad, ei_i2u[1], dpad])

    zrow = jnp.zeros((CH, D), jnp.float32)
    ones = jnp.ones((CH, D), jnp.float32)

    xcat = jnp.concatenate([x_user, x_item], axis=0)
    W1m = jnp.stack([W1_u2i, W1_i2u])
    W1r = jnp.stack([W1_root_user, W1_root_item])
    W2m = jnp.stack([W2_u2i, W2_i2u])
    W2r = jnp.stack([W2_root_user, W2_root_item])

    m1, r1 = _mm(xcat, W1m, W1r)
    sums1, counts = _sc_kernel(True)(src_all, dst_all, m1, zrow, ones)
    h1 = _combine(sums1, counts, r1, act=True)
    m2, r2 = _mm(h1, W2m, W2r)
    (sums2,) = _sc_kernel(False)(src_all, dst_all, m2, zrow, ones)
    out = _combine(sums2, counts, r2, act=False)
    return out
